# baseline (speedup 1.0000x reference)
import jax
import jax.numpy as jnp
from jax import lax
from jax.experimental import pallas as pl
from jax.experimental.pallas import tpu as pltpu
from jax.experimental.pallas import tpu_sc as plsc

NUM_PID = 5532
THRESHOLD = 0.2
LUT_DIM = 256
BATCH = 16384

A_BLK = 512
A_GRID = (NUM_PID + A_BLK - 1) // A_BLK
NP_PAD = A_GRID * A_BLK

B_BLK = 512
B_GRID = BATCH // (B_BLK * 2)

SC_CORES = 2
SC_SUBCORES = 16
SC_LANES = 16
SC_WORKERS = SC_CORES * SC_SUBCORES
B_PER_W = BATCH // SC_WORKERS
NEG_BIG = -1e30


def _stream_half(x, lbl):
    col = lax.broadcasted_iota(jnp.int32, (B_BLK, NUM_PID), 1)
    m = jnp.max(x, axis=1)
    s = jnp.sum(jnp.exp(x - m[:, None]), axis=1)
    g = jnp.minimum(lbl, NUM_PID - 1)
    picked = jnp.sum(jnp.where(col == g[:, None], x, 0.0), axis=1)
    return m, s, picked


def _stream_kernel(lut_blk_ref, lut_t_ref, logits0_ref, logits1_ref,
                   label0_ref, label1_ref,
                   maxval_ref, maxind_ref, stats0_ref, stats1_ref):
    i = pl.program_id(0)
    sim = jax.lax.dot_general(
        lut_blk_ref[...], lut_t_ref[...],
        (((1,), (0,)), ((), ())),
        preferred_element_type=jnp.float32,
    )
    blk = jnp.minimum(i, A_GRID - 1)
    row = blk * A_BLK + lax.broadcasted_iota(jnp.int32, (A_BLK, NUM_PID), 0)
    colA = lax.broadcasted_iota(jnp.int32, (A_BLK, NUM_PID), 1)
    sim = jnp.where(colA == row, 0.0, sim)
    mv = jnp.max(sim, axis=1)
    ind = jnp.min(jnp.where(sim == mv[:, None], colA, NUM_PID), axis=1)
    maxval_ref[...] = mv
    maxind_ref[...] = ind

    m0, s0, p0 = _stream_half(logits0_ref[0], label0_ref[0, 0, :])
    stats0_ref[0, 0, :] = m0
    stats0_ref[0, 1, :] = s0
    stats0_ref[0, 2, :] = p0
    m1, s1, p1 = _stream_half(logits1_ref[0], label1_ref[0, 0, :])
    stats1_ref[0, 0, :] = m1
    stats1_ref[0, 1, :] = s1
    stats1_ref[0, 2, :] = p1


def _fix_sc_kernel(label_hbm, maxval_hbm, maxind_hbm, logits8_hbm, fix_hbm,
                   label_v, maxval_v, maxind_v,
                   idxq_v, rem_v, ig_v, rows_v, fix_v, sem):
    wid = lax.axis_index("s") * SC_CORES + lax.axis_index("c")
    base = wid * B_PER_W
    pltpu.sync_copy(label_hbm.at[pl.ds(base, B_PER_W)], label_v)
    pltpu.sync_copy(maxval_hbm, maxval_v)
    pltpu.sync_copy(maxind_hbm, maxind_v)

    def body(j, _):
        off = j * SC_LANES
        lbl = label_v[pl.ds(off, SC_LANES)]
        g = jnp.minimum(lbl, NUM_PID - 1)
        val = plsc.load_gather(maxval_v, [g])
        ind = plsc.load_gather(maxind_v, [g])
        ignore = (lbl < NUM_PID) & (val > THRESHOLD)
        r = base + off + lax.iota(jnp.int32, SC_LANES)
        vidx = jnp.where(ignore, r * NUM_PID + ind, 0)
        idxq_v[pl.ds(off, SC_LANES)] = lax.shift_right_logical(vidx, 7)
        rem_v[pl.ds(off, SC_LANES)] = jnp.bitwise_and(vidx, 127)
        ig_v[pl.ds(off, SC_LANES)] = jnp.where(ignore, 1, 0)
        return 0

    lax.fori_loop(0, B_PER_W // SC_LANES, body, 0)

    copies = []
    for k in range(B_PER_W // 128):
        copies.append(pltpu.async_copy(
            logits8_hbm.at[idxq_v.at[pl.ds(k * 128, 128)]],
            rows_v.at[pl.ds(k * 128, 128)], sem))
    for c in copies:
        c.wait()

    def body2(j, _):
        off = j * SC_LANES
        lrow = off + lax.iota(jnp.int32, SC_LANES)
        v = plsc.load_gather(rows_v, [lrow, rem_v[pl.ds(off, SC_LANES)]])
        ig = ig_v[pl.ds(off, SC_LANES)]
        fix_v[pl.ds(off, SC_LANES)] = jnp.where(ig != 0, v, NEG_BIG)
        return 0

    lax.fori_loop(0, B_PER_W // SC_LANES, body2, 0)
    pltpu.sync_copy(fix_v, fix_hbm.at[pl.ds(base, B_PER_W)])


def _fix_values(label, maxval, maxind, logits):
    mesh = plsc.VectorSubcoreMesh(core_axis_name="c", subcore_axis_name="s")
    k = pl.kernel(
        _fix_sc_kernel,
        mesh=mesh,
        out_type=jax.ShapeDtypeStruct((BATCH,), jnp.float32),
        scratch_types=[
            pltpu.VMEM((B_PER_W,), jnp.int32),
            pltpu.VMEM((NP_PAD,), jnp.float32),
            pltpu.VMEM((NP_PAD,), jnp.int32),
            pltpu.VMEM((B_PER_W,), jnp.int32),
            pltpu.VMEM((B_PER_W,), jnp.int32),
            pltpu.VMEM((B_PER_W,), jnp.int32),
            pltpu.VMEM((B_PER_W, 128), jnp.float32),
            pltpu.VMEM((B_PER_W,), jnp.float32),
            pltpu.SemaphoreType.DMA,
        ],
        compiler_params=pltpu.CompilerParams(needs_layout_passes=False),
    )
    logits8 = logits.reshape(BATCH * NUM_PID // 128, 128)
    return k(label, maxval, maxind, logits8)


def _combine_kernel(stats0_ref, stats1_ref, fix_ref, label_ref, out_ref):
    acc = jnp.zeros((B_BLK,), jnp.float32)
    cnt = jnp.zeros((B_BLK,), jnp.float32)
    for half in range(2):
        stats = stats0_ref if half == 0 else stats1_ref
        for i in range(B_GRID):
            m = stats[i, 0, :]
            s = stats[i, 1, :]
            picked = stats[i, 2, :]
            blk = half * B_GRID + i
            fx = fix_ref[blk, 0, :]
            lbl = label_ref[blk, 0, :]
            valid = (lbl < NUM_PID).astype(jnp.float32)
            s2 = s - jnp.exp(fx - m) + jnp.where(fx > -1e29,
                                                 jnp.exp(-m), 0.0)
            lse = m + jnp.log(s2)
            acc = acc + (picked - lse) * valid
            cnt = cnt + valid
    out_ref[0, :] = jnp.broadcast_to(jnp.sum(acc), (B_BLK,))
    out_ref[1, :] = jnp.broadcast_to(jnp.sum(cnt), (B_BLK,))


def _combine(stats0, stats1, fix, label):
    nblk = BATCH // B_BLK
    fix3 = fix.reshape(nblk, 1, B_BLK)
    label3 = label.reshape(nblk, 1, B_BLK)
    return pl.pallas_call(
        _combine_kernel,
        grid=(1,),
        in_specs=[
            pl.BlockSpec((B_GRID, 8, B_BLK), lambda i: (0, 0, 0)),
            pl.BlockSpec((B_GRID, 8, B_BLK), lambda i: (0, 0, 0)),
            pl.BlockSpec((nblk, 1, B_BLK), lambda i: (0, 0, 0)),
            pl.BlockSpec((nblk, 1, B_BLK), lambda i: (0, 0, 0)),
        ],
        out_specs=pl.BlockSpec((8, B_BLK), lambda i: (0, 0)),
        out_shape=jax.ShapeDtypeStruct((8, B_BLK), jnp.float32),
    )(stats0, stats1, fix3, label3)


def kernel(logits, label, lut):
    label = label.astype(jnp.int32)
    maxval, maxind, stats0, stats1 = _stream_stats_call(logits, label, lut)
    fix = _fix_values(label, maxval, maxind, logits)
    out = _combine(stats0, stats1, fix, label)
    num = out[0, 0]
    cnt = jnp.maximum(out[1, 0], 1.0)
    return -num / cnt


def _stream_stats_call(logits, label, lut):
    nblk = BATCH // B_BLK
    label3 = label.reshape(nblk, 1, B_BLK)
    lg = logits.reshape(2, BATCH // 2, NUM_PID)
    return pl.pallas_call(
        _stream_kernel,
        grid=(B_GRID,),
        in_specs=[
            pl.BlockSpec((A_BLK, LUT_DIM),
                         lambda i: (jnp.minimum(i, A_GRID - 1), 0)),
            pl.BlockSpec((LUT_DIM, NUM_PID), lambda i: (0, 0)),
            pl.BlockSpec((1, B_BLK, NUM_PID), lambda i: (0, i, 0)),
            pl.BlockSpec((1, B_BLK, NUM_PID), lambda i: (1, i, 0)),
            pl.BlockSpec((1, 1, B_BLK), lambda i: (i, 0, 0)),
            pl.BlockSpec((1, 1, B_BLK), lambda i: (i + B_GRID, 0, 0)),
        ],
        out_specs=[
            pl.BlockSpec((A_BLK,), lambda i: (jnp.minimum(i, A_GRID - 1),)),
            pl.BlockSpec((A_BLK,), lambda i: (jnp.minimum(i, A_GRID - 1),)),
            pl.BlockSpec((1, 8, B_BLK), lambda i: (i, 0, 0)),
            pl.BlockSpec((1, 8, B_BLK), lambda i: (i, 0, 0)),
        ],
        out_shape=[
            jax.ShapeDtypeStruct((NP_PAD,), jnp.float32),
            jax.ShapeDtypeStruct((NP_PAD,), jnp.int32),
            jax.ShapeDtypeStruct((B_GRID, 8, B_BLK), jnp.float32),
            jax.ShapeDtypeStruct((B_GRID, 8, B_BLK), jnp.float32),
        ],
        compiler_params=pltpu.CompilerParams(
            vmem_limit_bytes=100 * 1024 * 1024),
    )(lut, lut.T, lg, lg, label3, label3)


# speedup vs baseline: 2.5561x; 2.5561x over previous
import functools

import jax
import jax.numpy as jnp
from jax import lax
from jax.experimental import pallas as pl
from jax.experimental.pallas import tpu as pltpu
from jax.experimental.pallas import tpu_sc as plsc

NUM_PID = 5532
THRESHOLD = 0.2
LUT_DIM = 256
BATCH = 16384

A_BLK = 512
A_GRID = (NUM_PID + A_BLK - 1) // A_BLK
NP_PAD = A_GRID * A_BLK

B_BLK = 512
B_NSTREAM = 2
B_GRID = BATCH // (B_BLK * B_NSTREAM)
B_HALF_BLOCKS = BATCH // (B_BLK * B_NSTREAM)

SC_CORES = 2
SC_SUBCORES = 16
SC_LANES = 16
SC_WORKERS = SC_CORES * SC_SUBCORES
B_PER_W = BATCH // SC_WORKERS


def _simmax_kernel(lut_blk_ref, lut_full_ref, maxval_ref, maxind_ref):
    i = pl.program_id(0)
    sim = jax.lax.dot_general(
        lut_blk_ref[...], lut_full_ref[...],
        (((1,), (1,)), ((), ())),
        preferred_element_type=jnp.float32,
    )
    row = i * A_BLK + lax.broadcasted_iota(jnp.int32, (A_BLK, NUM_PID), 0)
    col = lax.broadcasted_iota(jnp.int32, (A_BLK, NUM_PID), 1)
    sim = jnp.where(col == row, 0.0, sim)
    m = jnp.max(sim, axis=1)
    ind = jnp.min(jnp.where(sim == m[:, None], col, NUM_PID), axis=1)
    maxval_ref[...] = m
    maxind_ref[...] = ind


def _sim_max_argmax(lut):
    return pl.pallas_call(
        _simmax_kernel,
        grid=(A_GRID,),
        in_specs=[
            pl.BlockSpec((A_BLK, LUT_DIM), lambda i: (i, 0)),
            pl.BlockSpec((NUM_PID, LUT_DIM), lambda i: (0, 0)),
        ],
        out_specs=[
            pl.BlockSpec((A_BLK,), lambda i: (i,)),
            pl.BlockSpec((A_BLK,), lambda i: (i,)),
        ],
        out_shape=[
            jax.ShapeDtypeStruct((NP_PAD,), jnp.float32),
            jax.ShapeDtypeStruct((NP_PAD,), jnp.int32),
        ],
    )(lut, lut)


def _killcol_sc_kernel(label_hbm, maxval_hbm, maxind_hbm, kill_hbm,
                       label_v, maxval_v, maxind_v, kill_v):
    wid = lax.axis_index("s") * SC_CORES + lax.axis_index("c")
    base = wid * B_PER_W
    pltpu.sync_copy(label_hbm.at[pl.ds(base, B_PER_W)], label_v)
    pltpu.sync_copy(maxval_hbm, maxval_v)
    pltpu.sync_copy(maxind_hbm, maxind_v)

    def body(j, _):
        off = j * SC_LANES
        lbl = label_v[pl.ds(off, SC_LANES)]
        g = jnp.minimum(lbl, NUM_PID - 1)
        val = plsc.load_gather(maxval_v, [g])
        ind = plsc.load_gather(maxind_v, [g])
        ignore = (lbl < NUM_PID) & (val > THRESHOLD)
        kill_v[pl.ds(off, SC_LANES)] = jnp.where(ignore, ind, -1)
        return 0

    lax.fori_loop(0, B_PER_W // SC_LANES, body, 0)
    pltpu.sync_copy(kill_v, kill_hbm.at[wid, 0])


@functools.partial(jax.jit, static_argnames=())
def _kill_cols(label, maxval, maxind):
    mesh = plsc.VectorSubcoreMesh(core_axis_name="c", subcore_axis_name="s")
    k = pl.kernel(
        _killcol_sc_kernel,
        mesh=mesh,
        out_type=jax.ShapeDtypeStruct((SC_WORKERS, 1, B_PER_W), jnp.int32),
        scratch_types=[
            pltpu.VMEM((B_PER_W,), jnp.int32),
            pltpu.VMEM((NP_PAD,), jnp.float32),
            pltpu.VMEM((NP_PAD,), jnp.int32),
            pltpu.VMEM((B_PER_W,), jnp.int32),
        ],
        compiler_params=pltpu.CompilerParams(needs_layout_passes=False),
    )
    return k(label, maxval, maxind)


def _stream_part(x, lbl, kill):
    col = lax.broadcasted_iota(jnp.int32, (B_BLK, NUM_PID), 1)
    xm = jnp.where(col == kill[:, None], 0.0, x)
    m = jnp.max(xm, axis=1)
    s = jnp.sum(jnp.exp(xm - m[:, None]), axis=1)
    g = jnp.minimum(lbl, NUM_PID - 1)
    picked = jnp.sum(jnp.where(col == g[:, None], xm, 0.0), axis=1)
    valid = (lbl < NUM_PID).astype(jnp.float32)
    per_row = (picked - m - jnp.log(s)) * valid
    return jnp.sum(per_row), jnp.sum(valid)


def _loss_kernel(logits0_ref, logits1_ref, label0_ref, label1_ref,
                 kill0_ref, kill1_ref, out_ref):
    i = pl.program_id(0)
    p0, c0 = _stream_part(logits0_ref[0], label0_ref[0, 0, :],
                          kill0_ref[0, 0, :])
    p1, c1 = _stream_part(logits1_ref[0], label1_ref[0, 0, :],
                          kill1_ref[0, 0, :])
    part = p0 + p1
    cnt = c0 + c1

    @pl.when(i == 0)
    def _():
        out_ref[...] = jnp.zeros_like(out_ref)

    r8 = lax.broadcasted_iota(jnp.int32, (8, 128), 0)
    c8 = lax.broadcasted_iota(jnp.int32, (8, 128), 1)
    acc = jnp.where((r8 == 0) & (c8 == 0), part,
                    jnp.where((r8 == 0) & (c8 == 1), cnt, 0.0))
    out_ref[...] += acc


def _stream_loss(logits, label, kill):
    nblk = BATCH // B_BLK
    label3 = label.reshape(nblk, 1, B_BLK)
    kill3 = kill
    lg = logits.reshape(B_NSTREAM, BATCH // B_NSTREAM, NUM_PID)
    out = pl.pallas_call(
        _loss_kernel,
        grid=(B_GRID,),
        in_specs=[
            pl.BlockSpec((1, B_BLK, NUM_PID), lambda i: (0, i, 0)),
            pl.BlockSpec((1, B_BLK, NUM_PID), lambda i: (1, i, 0)),
            pl.BlockSpec((1, 1, B_BLK), lambda i: (i, 0, 0)),
            pl.BlockSpec((1, 1, B_BLK), lambda i: (i + B_GRID, 0, 0)),
            pl.BlockSpec((1, 1, B_BLK), lambda i: (i, 0, 0)),
            pl.BlockSpec((1, 1, B_BLK), lambda i: (i + B_GRID, 0, 0)),
        ],
        out_specs=pl.BlockSpec((8, 128), lambda i: (0, 0)),
        out_shape=jax.ShapeDtypeStruct((8, 128), jnp.float32),
    )(lg, lg, label3, label3, kill3, kill3)
    return out


def kernel(logits, label, lut):
    label = label.astype(jnp.int32)
    maxval, maxind = _sim_max_argmax(lut)
    kill = _kill_cols(label, maxval, maxind)
    out = _stream_loss(logits, label, kill)
    num = out[0, 0]
    cnt = jnp.maximum(out[0, 1], 1.0)
    return -num / cnt


# speedup vs baseline: 2.5829x; 1.0105x over previous
import functools

import jax
import jax.numpy as jnp
from jax import lax
from jax.experimental import pallas as pl
from jax.experimental.pallas import tpu as pltpu
from jax.experimental.pallas import tpu_sc as plsc

NUM_PID = 5532
THRESHOLD = 0.2
LUT_DIM = 256
BATCH = 16384

A_BLK = 512
A_GRID = (NUM_PID + A_BLK - 1) // A_BLK
NP_PAD = A_GRID * A_BLK

B_BLK = 512
B_NSTREAM = 2
B_GRID = BATCH // (B_BLK * B_NSTREAM)
B_HALF_BLOCKS = BATCH // (B_BLK * B_NSTREAM)

SC_CORES = 2
SC_SUBCORES = 16
SC_LANES = 16
SC_WORKERS = SC_CORES * SC_SUBCORES
B_PER_W = BATCH // SC_WORKERS


def _simmax_kernel(lut_blk_ref, lut_full_ref, maxval_ref, maxind_ref):
    i = pl.program_id(0)
    sim = jax.lax.dot_general(
        lut_blk_ref[...], lut_full_ref[...],
        (((1,), (1,)), ((), ())),
        preferred_element_type=jnp.float32,
    )
    row = i * A_BLK + lax.broadcasted_iota(jnp.int32, (A_BLK, NUM_PID), 0)
    col = lax.broadcasted_iota(jnp.int32, (A_BLK, NUM_PID), 1)
    sim = jnp.where(col == row, 0.0, sim)
    m = jnp.max(sim, axis=1)
    ind = jnp.min(jnp.where(sim == m[:, None], col, NUM_PID), axis=1)
    maxval_ref[...] = m
    maxind_ref[...] = ind


def _sim_max_argmax(lut):
    return pl.pallas_call(
        _simmax_kernel,
        grid=(A_GRID,),
        in_specs=[
            pl.BlockSpec((A_BLK, LUT_DIM), lambda i: (i, 0)),
            pl.BlockSpec((NUM_PID, LUT_DIM), lambda i: (0, 0)),
        ],
        out_specs=[
            pl.BlockSpec((A_BLK,), lambda i: (i,)),
            pl.BlockSpec((A_BLK,), lambda i: (i,)),
        ],
        out_shape=[
            jax.ShapeDtypeStruct((NP_PAD,), jnp.float32),
            jax.ShapeDtypeStruct((NP_PAD,), jnp.int32),
        ],
    )(lut, lut)


def _killcol_sc_kernel(label_hbm, maxval_hbm, maxind_hbm, kill_hbm,
                       label_v, maxval_v, maxind_v, kill_v):
    wid = lax.axis_index("s") * SC_CORES + lax.axis_index("c")
    base = wid * B_PER_W
    pltpu.sync_copy(label_hbm.at[pl.ds(base, B_PER_W)], label_v)
    pltpu.sync_copy(maxval_hbm, maxval_v)
    pltpu.sync_copy(maxind_hbm, maxind_v)

    def body(j, _):
        off = j * SC_LANES
        lbl = label_v[pl.ds(off, SC_LANES)]
        g = jnp.minimum(lbl, NUM_PID - 1)
        val = plsc.load_gather(maxval_v, [g])
        ind = plsc.load_gather(maxind_v, [g])
        ignore = (lbl < NUM_PID) & (val > THRESHOLD)
        kill_v[pl.ds(off, SC_LANES)] = jnp.where(ignore, ind, -1)
        return 0

    lax.fori_loop(0, B_PER_W // SC_LANES, body, 0)
    pltpu.sync_copy(kill_v, kill_hbm.at[wid, 0])


@functools.partial(jax.jit, static_argnames=())
def _kill_cols(label, maxval, maxind):
    mesh = plsc.VectorSubcoreMesh(core_axis_name="c", subcore_axis_name="s")
    k = pl.kernel(
        _killcol_sc_kernel,
        mesh=mesh,
        out_type=jax.ShapeDtypeStruct((SC_WORKERS, 1, B_PER_W), jnp.int32),
        scratch_types=[
            pltpu.VMEM((B_PER_W,), jnp.int32),
            pltpu.VMEM((NP_PAD,), jnp.float32),
            pltpu.VMEM((NP_PAD,), jnp.int32),
            pltpu.VMEM((B_PER_W,), jnp.int32),
        ],
        compiler_params=pltpu.CompilerParams(needs_layout_passes=False),
    )
    return k(label, maxval, maxind)


def _stream_part(x, lbl, kill):
    col = lax.broadcasted_iota(jnp.int32, (B_BLK, NUM_PID), 1)
    xm = jnp.where(col == kill[:, None], 0.0, x)
    m = jnp.max(xm, axis=1)
    s = jnp.sum(jnp.exp(xm - m[:, None]), axis=1)
    g = jnp.minimum(lbl, NUM_PID - 1)
    picked = jnp.sum(jnp.where(col == g[:, None], xm, 0.0), axis=1)
    valid = (lbl < NUM_PID).astype(jnp.float32)
    per_row = (picked - m - jnp.log(s)) * valid
    return per_row.reshape(B_BLK // 128, 128), valid.reshape(B_BLK // 128, 128)


def _loss_kernel(logits0_ref, logits1_ref, label0_ref, label1_ref,
                 kill0_ref, kill1_ref, out_ref):
    i = pl.program_id(0)
    p0, c0 = _stream_part(logits0_ref[0], label0_ref[0, 0, :],
                          kill0_ref[0, 0, :])
    p1, c1 = _stream_part(logits1_ref[0], label1_ref[0, 0, :],
                          kill1_ref[0, 0, :])

    @pl.when(i == 0)
    def _():
        out_ref[...] = jnp.zeros_like(out_ref)

    out_ref[0:4, :] += p0 + p1
    out_ref[4:8, :] += c0 + c1


def _stream_loss(logits, label, kill):
    nblk = BATCH // B_BLK
    label3 = label.reshape(nblk, 1, B_BLK)
    kill3 = kill
    lg = logits.reshape(B_NSTREAM, BATCH // B_NSTREAM, NUM_PID)
    out = pl.pallas_call(
        _loss_kernel,
        grid=(B_GRID,),
        in_specs=[
            pl.BlockSpec((1, B_BLK, NUM_PID), lambda i: (0, i, 0)),
            pl.BlockSpec((1, B_BLK, NUM_PID), lambda i: (1, i, 0)),
            pl.BlockSpec((1, 1, B_BLK), lambda i: (i, 0, 0)),
            pl.BlockSpec((1, 1, B_BLK), lambda i: (i + B_GRID, 0, 0)),
            pl.BlockSpec((1, 1, B_BLK), lambda i: (i, 0, 0)),
            pl.BlockSpec((1, 1, B_BLK), lambda i: (i + B_GRID, 0, 0)),
        ],
        out_specs=pl.BlockSpec((8, 128), lambda i: (0, 0)),
        out_shape=jax.ShapeDtypeStruct((8, 128), jnp.float32),
    )(lg, lg, label3, label3, kill3, kill3)
    return out


def kernel(logits, label, lut):
    label = label.astype(jnp.int32)
    maxval, maxind = _sim_max_argmax(lut)
    kill = _kill_cols(label, maxval, maxind)
    out = _stream_loss(logits, label, kill)
    num = jnp.sum(out[0:4, :])
    cnt = jnp.maximum(jnp.sum(out[4:8, :]), 1.0)
    return -num / cnt
